# baseline (speedup 1.0000x reference)
import jax
import jax.numpy as jnp
from jax import lax
from jax.experimental import pallas as pl
from jax.experimental.pallas import tpu as pltpu
from jax.experimental.pallas import tpu_sc as plsc

_H = 32
_NH = 12
_NB = 4
_S = _H * _H
_PW = 32
_HS = 16
_NT = _NH * _NB * 2


def _sc_body(tf_hbm, x_hbm, out_hbm, table_v, bias_v, buf0, buf1, buf2,
             si0, si1, si2, so0, so1, so2):
    w = lax.axis_index("s") * 2 + lax.axis_index("c")
    bufs = (buf0, buf1, buf2)
    sin = (si0, si1, si2)
    sout = (so0, so1, so2)

    def x_slab(t):
        b = (t // 2) % _NB
        h = t // (2 * _NB)
        return x_hbm.at[b, h, pl.ds(w * _PW + (t % 2) * _HS, _HS)]

    def out_slab(t):
        b = (t // 2) % _NB
        h = t // (2 * _NB)
        return out_hbm.at[b, h, pl.ds(w * _PW + (t % 2) * _HS, _HS)]

    pltpu.async_copy(x_slab(0), bufs[0], sin[0])
    pltpu.async_copy(x_slab(1), bufs[1], sin[1])

    def super_body(s, carry):
        for k in range(3):
            t = 3 * s + k
            h = t // (2 * _NB)

            @pl.when(t % (2 * _NB) == 0)
            def _new_head():
                pltpu.sync_copy(tf_hbm.at[h], table_v)

                def build_body(pj, c):
                    for qi in range(_H):
                        a = w + (_H - 1) - qi
                        bias_v[pj, pl.ds(qi * _H, 16)] = \
                            table_v[a, pl.ds(_H - 1 - pj, 16)]
                        bias_v[pj, pl.ds(qi * _H + 16, 16)] = \
                            table_v[a, pl.ds(_H - 1 - pj + 16, 16)]
                    return c

                lax.fori_loop(0, _PW, build_body, 0)

            pltpu.make_async_copy(x_slab(0), bufs[k], sin[k]).wait()
            hs16 = (t % 2) * _HS

            def add_body(row, c):
                buf = bufs[k]
                br = hs16 + row
                for cc in range(_S // 16):
                    sl = pl.ds(cc * 16, 16)
                    buf[row, sl] = buf[row, sl] + bias_v[br, sl]
                return c

            lax.fori_loop(0, _HS, add_body, 0)
            pltpu.async_copy(bufs[k], out_slab(t), sout[k])

            kn = (k + 2) % 3

            @pl.when(t + 2 < _NT)
            def _prefetch():
                @pl.when(t >= 1)
                def _drain_prev_out():
                    pltpu.make_async_copy(bufs[kn], out_slab(0), sout[kn]).wait()
                pltpu.async_copy(x_slab(t + 2), bufs[kn], sin[kn])
        return carry

    lax.fori_loop(0, _NT // 3, super_body, 0)
    for k in range(3):
        pltpu.make_async_copy(bufs[k], out_slab(0), sout[k]).wait()


def kernel(x, relative_pos):
    tf = jnp.pad(relative_pos[:, :, ::-1], ((0, 0), (0, 1), (0, 1)))
    mesh = plsc.VectorSubcoreMesh(core_axis_name="c", subcore_axis_name="s",
                                  num_cores=2)
    run = pl.kernel(
        _sc_body,
        mesh=mesh,
        out_type=jax.ShapeDtypeStruct(x.shape, x.dtype),
        scratch_types=[
            pltpu.VMEM((64, 64), jnp.float32),
            pltpu.VMEM((_PW, _S), jnp.float32),
            pltpu.VMEM((_HS, _S), jnp.float32),
            pltpu.VMEM((_HS, _S), jnp.float32),
            pltpu.VMEM((_HS, _S), jnp.float32),
            pltpu.SemaphoreType.DMA,
            pltpu.SemaphoreType.DMA,
            pltpu.SemaphoreType.DMA,
            pltpu.SemaphoreType.DMA,
            pltpu.SemaphoreType.DMA,
            pltpu.SemaphoreType.DMA,
        ],
    )
    return run(tf, x)


# speedup vs baseline: 1.5423x; 1.5423x over previous
import jax
import jax.numpy as jnp
from jax import lax
from jax.experimental import pallas as pl
from jax.experimental.pallas import tpu as pltpu
from jax.experimental.pallas import tpu_sc as plsc

_H = 32
_NH = 12
_NB = 4
_S = _H * _H
_PW = 32
_HS = 16
_NG = _NH * 2


def _sc_body(tf_hbm, x_hbm, out_hbm, table_v, bias_v,
             buf0, buf1, buf2, buf3, si0, si1, si2, si3,
             so0, so1, so2, so3):
    w = lax.axis_index("s") * 2 + lax.axis_index("c")
    bufs = (buf0, buf1, buf2, buf3)
    sin = (si0, si1, si2, si3)
    sout = (so0, so1, so2, so3)

    def g_body(t, carry):
        h = t // 2
        hs = t % 2
        row0 = w * _PW + hs * _HS
        for b in range(_NB):
            pltpu.async_copy(x_hbm.at[b, h, pl.ds(row0, _HS)], bufs[b], sin[b])
        pltpu.sync_copy(tf_hbm.at[h], table_v)

        def build_body(rl, c):
            pj = hs * _HS + rl
            for qi in range(_H):
                a = w + (_H - 1) - qi
                bias_v[rl, pl.ds(qi * _H, 16)] = \
                    table_v[a, pl.ds(_H - 1 - pj, 16)]
                bias_v[rl, pl.ds(qi * _H + 16, 16)] = \
                    table_v[a, pl.ds(_H - 1 - pj + 16, 16)]
            return c

        lax.fori_loop(0, _HS, build_body, 0)

        for b in range(_NB):
            pltpu.make_async_copy(x_hbm.at[b, h, pl.ds(row0, _HS)],
                                  bufs[b], sin[b]).wait()

            def add_body(row, c):
                buf = bufs[b]
                for cc in range(_S // 16):
                    sl = pl.ds(cc * 16, 16)
                    buf[row, sl] = buf[row, sl] + bias_v[row, sl]
                return c

            lax.fori_loop(0, _HS, add_body, 0)
            pltpu.async_copy(bufs[b], out_hbm.at[b, h, pl.ds(row0, _HS)],
                             sout[b])
        for b in range(_NB):
            pltpu.make_async_copy(bufs[b], out_hbm.at[b, h, pl.ds(row0, _HS)],
                                  sout[b]).wait()
        return carry

    lax.fori_loop(0, _NG, g_body, 0)


def kernel(x, relative_pos):
    tf = jnp.pad(relative_pos[:, :, ::-1], ((0, 0), (0, 1), (0, 1)))
    mesh = plsc.VectorSubcoreMesh(core_axis_name="c", subcore_axis_name="s",
                                  num_cores=2)
    run = pl.kernel(
        _sc_body,
        mesh=mesh,
        out_type=jax.ShapeDtypeStruct(x.shape, x.dtype),
        scratch_types=[
            pltpu.VMEM((64, 64), jnp.float32),
            pltpu.VMEM((_HS, _S), jnp.float32),
            pltpu.VMEM((_HS, _S), jnp.float32),
            pltpu.VMEM((_HS, _S), jnp.float32),
            pltpu.VMEM((_HS, _S), jnp.float32),
            pltpu.VMEM((_HS, _S), jnp.float32),
            pltpu.SemaphoreType.DMA,
            pltpu.SemaphoreType.DMA,
            pltpu.SemaphoreType.DMA,
            pltpu.SemaphoreType.DMA,
            pltpu.SemaphoreType.DMA,
            pltpu.SemaphoreType.DMA,
            pltpu.SemaphoreType.DMA,
            pltpu.SemaphoreType.DMA,
        ],
    )
    return run(tf, x)
